# baseline (speedup 1.0000x reference)
import functools

import jax
import jax.numpy as jnp
from jax import lax
from jax.experimental import pallas as pl
from jax.experimental.pallas import tpu as pltpu
from jax.experimental.pallas import tpu_sc as plsc

F = 26
E = 16
EOD = F * E
EPS = 1e-5


def _tc_flatten_body(t_ref, o_ref):
    nr = o_ref.shape[0]
    eye = jax.lax.broadcasted_iota(jnp.int32, (E, E), 0) == \
        jax.lax.broadcasted_iota(jnp.int32, (E, E), 1)
    tt = jax.lax.dot_general(
        t_ref[:], eye.astype(jnp.float32),
        (((0,), (0,)), ((), ())),
        preferred_element_type=jnp.float32,
    ).reshape(nr, 8, E)
    for s in range(8):
        o_ref[:, E * s:E * (s + 1)] = tt[:, s, :]


def _tc_flatten(embT, cols):
    v = embT.shape[1]
    nr = cols * E // 128
    return pl.pallas_call(
        _tc_flatten_body,
        grid=(pl.cdiv(v, cols),),
        in_specs=[pl.BlockSpec((E, cols), lambda i: (0, i))],
        out_specs=pl.BlockSpec((nr, 128), lambda i: (i, 0)),
        out_shape=jax.ShapeDtypeStruct((v * E // 128, 128), jnp.float32),
        compiler_params=pltpu.CompilerParams(
            dimension_semantics=("arbitrary",),
        ),
    )(embT)


def _sc_gather(xi, emb_table, lin_flat):
    info = plsc.get_sparse_core_info()
    nc, ns = info.num_cores, info.num_subcores
    nw = nc * ns
    bf = xi.shape[0]
    per_w = bf // nw
    nch = 4
    ch = per_w // nch
    mesh = plsc.VectorSubcoreMesh(core_axis_name="c", subcore_axis_name="s")

    @functools.partial(
        pl.kernel,
        mesh=mesh,
        out_type=(
            jax.ShapeDtypeStruct((bf, E), jnp.float32),
            jax.ShapeDtypeStruct((bf,), jnp.float32),
        ),
        scratch_types=[
            pltpu.VMEM((ch,), jnp.int32),
            pltpu.VMEM((ch, E), jnp.float32),
            pltpu.VMEM((ch,), jnp.float32),
            pltpu.SemaphoreType.DMA,
            pltpu.SemaphoreType.DMA,
        ],
        compiler_params=pltpu.CompilerParams(use_tc_tiling_on_sc=False),
    )
    def k(xi_hbm, emb_hbm, lin_hbm, out_e, out_l, idx_v, rows_v, lin_v, sem1, sem2):
        wid = lax.axis_index("s") * nc + lax.axis_index("c")
        base = wid * per_w
        for c in range(nch):
            off = base + c * ch
            pltpu.sync_copy(xi_hbm.at[pl.ds(off, ch)], idx_v)
            d1 = pltpu.async_copy(emb_hbm.at[idx_v], rows_v, sem1)
            d2 = pltpu.async_copy(lin_hbm.at[idx_v], lin_v, sem2)
            d1.wait()
            d2.wait()
            pltpu.sync_copy(rows_v, out_e.at[pl.ds(off, ch)])
            pltpu.sync_copy(lin_v, out_l.at[pl.ds(off, ch)])

    return k(xi, emb_table, lin_flat)


def _dense_body(g_ref, ling_ref, s0_ref, t0_ref, cw_ref, cb_ref, sexp_ref,
                w1_ref, c1_ref, w2_ref, c2_ref, w3_ref, b3_ref, lb_ref, o_ref):
    flat = g_ref[:] * s0_ref[:] + t0_ref[:]
    h = jnp.dot(flat, cw_ref[:], preferred_element_type=jnp.float32) + cb_ref[:]
    h = jnp.maximum(h, 0.0)
    m = jnp.max(h, axis=1, keepdims=True)
    p = jnp.exp(h - m)
    w = p / jnp.sum(p, axis=1, keepdims=True)
    z = flat * jnp.dot(w, sexp_ref[:], preferred_element_type=jnp.float32)
    a = jnp.dot(z, w1_ref[:], preferred_element_type=jnp.float32) + c1_ref[:]
    a = jnp.maximum(a, 0.0)
    a = jnp.dot(a, w2_ref[:], preferred_element_type=jnp.float32) + c2_ref[:]
    a = jnp.maximum(a, 0.0)
    zz = jnp.dot(a, w3_ref[:], preferred_element_type=jnp.float32) + b3_ref[:]
    lin = jnp.sum(ling_ref[:], axis=1, keepdims=True) + lb_ref[:]
    o_ref[:] = jax.nn.sigmoid(lin + zz)


def _dense_call(g, ling, s0, t0, cw, cb, sexp, w1, c1, w2, c2, w3, b3, lb, bm):
    b = g.shape[0]
    grid = (b // bm,)
    full = lambda shape: pl.BlockSpec(shape, lambda i: (0, 0))
    return pl.pallas_call(
        _dense_body,
        grid=grid,
        in_specs=[
            pl.BlockSpec((bm, EOD), lambda i: (i, 0)),
            pl.BlockSpec((bm, F), lambda i: (i, 0)),
            full((1, EOD)),
            full((1, EOD)),
            full((EOD, F)),
            full((1, F)),
            full((F, EOD)),
            full((EOD, 256)),
            full((1, 256)),
            full((256, 128)),
            full((1, 128)),
            full((128, 1)),
            full((1, 1)),
            full((1, 1)),
        ],
        out_specs=pl.BlockSpec((bm, 1), lambda i: (i, 0)),
        out_shape=jax.ShapeDtypeStruct((b, 1), jnp.float32),
        compiler_params=pltpu.CompilerParams(
            dimension_semantics=("arbitrary",),
        ),
    )(g, ling, s0, t0, cw, cb, sexp, w1, c1, w2, c2, w3, b3, lb)


def kernel(x, emb_table, lin_table, lin_bias, bn0_g, bn0_b, ctrl_W, ctrl_b,
           ctrl_bn_g, ctrl_bn_b, W1, b1, bn1_g, bn1_b, W2, b2, bn2_g, bn2_b,
           W3, b3):
    b = x.shape[0]
    inv = 1.0 / jnp.sqrt(jnp.float32(1.0 + EPS))
    offsets = (jnp.arange(F, dtype=x.dtype) * 100000)[None, :]
    xi = (x + offsets).reshape(-1)

    emb_g, lin_g = _sc_gather(xi, emb_table, lax.squeeze(lin_table, (1,)))
    g = emb_g.reshape(b, EOD)
    ling = lin_g.reshape(b, F)

    s0 = jnp.repeat(bn0_g * inv, E)[None, :]
    t0 = jnp.repeat(bn0_b, E)[None, :]
    cs = ctrl_bn_g * inv
    cw = ctrl_W * cs[None, :]
    cb = (ctrl_b * cs + ctrl_bn_b)[None, :]
    sexp = jnp.kron(jnp.eye(F, dtype=jnp.float32), jnp.ones((1, E), jnp.float32))
    s1 = bn1_g * inv
    w1 = W1 * s1[None, :]
    c1 = (b1 * s1 + bn1_b)[None, :]
    s2 = bn2_g * inv
    w2 = W2 * s2[None, :]
    c2 = (b2 * s2 + bn2_b)[None, :]

    out = _dense_call(g, ling, s0, t0, cw, cb, sexp, w1, c1, w2, c2, W3,
                      b3[None, :], lin_bias[None, :], bm=1024)
    return out.reshape(b)


# speedup vs baseline: 1.3792x; 1.3792x over previous
import functools

import jax
import jax.numpy as jnp
from jax import lax
from jax.experimental import pallas as pl
from jax.experimental.pallas import tpu as pltpu
from jax.experimental.pallas import tpu_sc as plsc

F = 26
E = 16
EOD = F * E
EPS = 1e-5


def _tc_flatten_body(t_ref, o_ref):
    nr = o_ref.shape[0]
    tt = t_ref[:].T.reshape(nr, 8, E)
    for s in range(8):
        o_ref[:, E * s:E * (s + 1)] = tt[:, s, :]


def _tc_flatten(embT, cols):
    v = embT.shape[1]
    nr = cols * E // 128
    return pl.pallas_call(
        _tc_flatten_body,
        grid=(pl.cdiv(v, cols),),
        in_specs=[pl.BlockSpec((E, cols), lambda i: (0, i))],
        out_specs=pl.BlockSpec((nr, 128), lambda i: (i, 0)),
        out_shape=jax.ShapeDtypeStruct((v * E // 128, 128), jnp.float32),
        compiler_params=pltpu.CompilerParams(
            dimension_semantics=("arbitrary",),
        ),
    )(embT)


def _sc_gather(xi, emb_table, lin_flat):
    info = plsc.get_sparse_core_info()
    nc, ns = info.num_cores, info.num_subcores
    nw = nc * ns
    bf = xi.shape[0]
    per_w = bf // nw
    nch = 4
    ch = per_w // nch
    mesh = plsc.VectorSubcoreMesh(core_axis_name="c", subcore_axis_name="s")

    @functools.partial(
        pl.kernel,
        mesh=mesh,
        out_type=(
            jax.ShapeDtypeStruct((bf, E), jnp.float32),
            jax.ShapeDtypeStruct((bf,), jnp.float32),
        ),
        scratch_types=[
            pltpu.VMEM((ch,), jnp.int32),
            pltpu.VMEM((ch, E), jnp.float32),
            pltpu.VMEM((ch,), jnp.float32),
            pltpu.SemaphoreType.DMA,
            pltpu.SemaphoreType.DMA,
        ],
        compiler_params=pltpu.CompilerParams(use_tc_tiling_on_sc=False),
    )
    def k(xi_hbm, emb_hbm, lin_hbm, out_e, out_l, idx_v, rows_v, lin_v, sem1, sem2):
        wid = lax.axis_index("s") * nc + lax.axis_index("c")
        base = wid * per_w
        for c in range(nch):
            off = base + c * ch
            pltpu.sync_copy(xi_hbm.at[pl.ds(off, ch)], idx_v)
            d1 = pltpu.async_copy(emb_hbm.at[idx_v], rows_v, sem1)
            d2 = pltpu.async_copy(lin_hbm.at[idx_v], lin_v, sem2)
            d1.wait()
            d2.wait()
            pltpu.sync_copy(rows_v, out_e.at[pl.ds(off, ch)])
            pltpu.sync_copy(lin_v, out_l.at[pl.ds(off, ch)])

    return k(xi, emb_table, lin_flat)


def _dense_body(g_ref, ling_ref, s0_ref, t0_ref, cw_ref, cb_ref, sexp_ref,
                w1_ref, c1_ref, w2_ref, c2_ref, w3_ref, b3_ref, lb_ref, o_ref):
    flat = g_ref[:] * s0_ref[:] + t0_ref[:]
    h = jnp.dot(flat, cw_ref[:], preferred_element_type=jnp.float32) + cb_ref[:]
    h = jnp.maximum(h, 0.0)
    m = jnp.max(h, axis=1, keepdims=True)
    p = jnp.exp(h - m)
    w = p / jnp.sum(p, axis=1, keepdims=True)
    z = flat * jnp.dot(w, sexp_ref[:], preferred_element_type=jnp.float32)
    a = jnp.dot(z, w1_ref[:], preferred_element_type=jnp.float32) + c1_ref[:]
    a = jnp.maximum(a, 0.0)
    a = jnp.dot(a, w2_ref[:], preferred_element_type=jnp.float32) + c2_ref[:]
    a = jnp.maximum(a, 0.0)
    zz = jnp.dot(a, w3_ref[:], preferred_element_type=jnp.float32) + b3_ref[:]
    lin = jnp.sum(ling_ref[:], axis=1, keepdims=True) + lb_ref[:]
    o_ref[:] = jax.nn.sigmoid(lin + zz)


def _dense_call(g, ling, s0, t0, cw, cb, sexp, w1, c1, w2, c2, w3, b3, lb, bm):
    b = g.shape[0]
    grid = (b // bm,)
    full = lambda shape: pl.BlockSpec(shape, lambda i: (0, 0))
    return pl.pallas_call(
        _dense_body,
        grid=grid,
        in_specs=[
            pl.BlockSpec((bm, EOD), lambda i: (i, 0)),
            pl.BlockSpec((bm, F), lambda i: (i, 0)),
            full((1, EOD)),
            full((1, EOD)),
            full((EOD, F)),
            full((1, F)),
            full((F, EOD)),
            full((EOD, 256)),
            full((1, 256)),
            full((256, 128)),
            full((1, 128)),
            full((128, 1)),
            full((1, 1)),
            full((1, 1)),
        ],
        out_specs=pl.BlockSpec((bm, 1), lambda i: (i, 0)),
        out_shape=jax.ShapeDtypeStruct((b, 1), jnp.float32),
        compiler_params=pltpu.CompilerParams(
            dimension_semantics=("arbitrary",),
        ),
    )(g, ling, s0, t0, cw, cb, sexp, w1, c1, w2, c2, w3, b3, lb)


def kernel(x, emb_table, lin_table, lin_bias, bn0_g, bn0_b, ctrl_W, ctrl_b,
           ctrl_bn_g, ctrl_bn_b, W1, b1, bn1_g, bn1_b, W2, b2, bn2_g, bn2_b,
           W3, b3):
    b = x.shape[0]
    inv = 1.0 / jnp.sqrt(jnp.float32(1.0 + EPS))
    offsets = (jnp.arange(F, dtype=x.dtype) * 100000)[None, :]
    xi = (x + offsets).reshape(-1)

    flat2d = _tc_flatten(emb_table.T, cols=20480)
    tbl = flat2d.reshape(-1).reshape(-1, E)
    emb_g, lin_g = _sc_gather(xi, tbl, lax.squeeze(lin_table, (1,)))
    g = emb_g.reshape(b, EOD)
    ling = lin_g.reshape(b, F)

    s0 = jnp.repeat(bn0_g * inv, E)[None, :]
    t0 = jnp.repeat(bn0_b, E)[None, :]
    cs = ctrl_bn_g * inv
    cw = ctrl_W * cs[None, :]
    cb = (ctrl_b * cs + ctrl_bn_b)[None, :]
    sexp = jnp.kron(jnp.eye(F, dtype=jnp.float32), jnp.ones((1, E), jnp.float32))
    s1 = bn1_g * inv
    w1 = W1 * s1[None, :]
    c1 = (b1 * s1 + bn1_b)[None, :]
    s2 = bn2_g * inv
    w2 = W2 * s2[None, :]
    c2 = (b2 * s2 + bn2_b)[None, :]

    out = _dense_call(g, ling, s0, t0, cw, cb, sexp, w1, c1, w2, c2, W3,
                      b3[None, :], lin_bias[None, :], bm=1024)
    return out.reshape(b)
